# baseline (speedup 1.0000x reference)
import dataclasses
import functools

import jax
import jax.numpy as jnp
from jax import lax
from jax.experimental import pallas as pl
from jax.experimental.pallas import tpu as pltpu
from jax.experimental.pallas import tpu_sc as plsc

N_NODES = 10000
D = 128
E_EDGES = 320000
NC = 2
NS = 16
NW = NC * NS
CE1, CPW1 = 64, 162
CE2, CPW2 = 80, 126
N_PAD = 10240
RPT = N_PAD // NS
CROWS = N_PAD // D

_NB = 3
_IS = 6
_SS = 2

_DOT = (((1,), (1,)), ((), ()))

_SC_PARAMS = dataclasses.replace(
    pltpu.CompilerParams(), needs_layout_passes=False)


def _make_seg_sum(with_counts, CE, CPW):
  mesh = plsc.VectorSubcoreMesh(core_axis_name="c", subcore_axis_name="s")
  out_type = [jax.ShapeDtypeStruct((NC, N_PAD, D), jnp.float32)]
  scratch = [
      pltpu.VMEM((_IS, CE), jnp.int32),
      pltpu.VMEM((_IS, CE), jnp.int32),
      pltpu.VMEM((_NB, CE, D), jnp.float32),
      pltpu.VMEM_SHARED((N_PAD, D), jnp.float32),
  ]
  if with_counts:
    out_type = out_type + [jax.ShapeDtypeStruct((NW, CROWS, D), jnp.float32)]
    scratch = scratch + [pltpu.VMEM((CROWS, D), jnp.float32)]
  n_out = 1 + int(with_counts)
  n_scr = len(scratch)

  @functools.partial(
      pl.kernel,
      out_type=out_type,
      mesh=mesh,
      scratch_types=scratch + [pltpu.SemaphoreType.DMA] * (2 * _IS + _NB + _SS),
      compiler_params=_SC_PARAMS,
  )
  def seg_sum(y_hbm, src_hbm, dst_hbm, *rest):
    out_hbm = rest[0]
    cnt_hbm = rest[1] if with_counts else None
    ivs, ivd, rows, acc = rest[n_out:n_out + 4]
    cnt = rest[n_out + 4] if with_counts else None
    sems = rest[n_out + n_scr:]
    sis = sems[:_IS]
    sid = sems[_IS:2 * _IS]
    sg = sems[2 * _IS:2 * _IS + _NB]
    ss = sems[2 * _IS + _NB:]
    c = lax.axis_index("c")
    s = lax.axis_index("s")
    wid = s * NC + c
    base_e = wid * (CPW * CE)

    def src_sl(i):
      return src_hbm.at[pl.ds(base_e + i * CE, CE)]

    def dst_sl(i):
      return dst_hbm.at[pl.ds(base_e + i * CE, CE)]

    def scat(u):
      return pltpu.make_async_copy(rows.at[u % _NB], acc.at[ivd.at[u % _IS]],
                                   ss[u % _SS])

    z16 = jnp.zeros((16,), jnp.float32)

    @pl.loop(0, CE)
    def _(r):
      for k16 in range(D // 16):
        rows[0, r, pl.ds(k16 * 16, 16)] = z16

    @pl.loop(0, RPT // CE)
    def _(j):
      pltpu.sync_copy(rows.at[0], acc.at[pl.ds(s * RPT + j * CE, CE)])

    if with_counts:
      @pl.loop(0, CROWS)
      def _(r):
        for k16 in range(D // 16):
          cnt[r, pl.ds(k16 * 16, 16)] = z16

    plsc.subcore_barrier()

    for u in range(_IS):
      pltpu.async_copy(src_sl(u), ivs.at[u], sis[u])
    for u in range(_IS - 1):
      pltpu.async_copy(dst_sl(u), ivd.at[u], sid[u])
    for u in range(2):
      pltpu.make_async_copy(src_sl(u), ivs.at[u], sis[u]).wait()
      pltpu.async_copy(y_hbm.at[ivs.at[u]], rows.at[u], sg[u])

    ones16 = jnp.ones((16,), jnp.float32)

    def row_slot(i, u, first, refill_s, refill_d, refill_g):
      pltpu.make_async_copy(y_hbm.at[ivs.at[u]], rows.at[u % _NB],
                            sg[u % _NB]).wait()
      pltpu.make_async_copy(dst_sl(i), ivd.at[u], sid[u]).wait()
      scat(u).start(add=True)
      if with_counts:
        for k16 in range(CE // 16):
          v = ivd[u, pl.ds(k16 * 16, 16)]
          plsc.addupdate_scatter(cnt, [v >> 7, v & 127], ones16)
      if refill_s:
        pltpu.async_copy(src_sl(i + _IS), ivs.at[u], sis[u])
      if not first:
        scat(u + _IS - 1).wait()
      if refill_d:
        u5 = (u + _IS - 1) % _IS
        pltpu.async_copy(dst_sl(i + _IS - 1), ivd.at[u5], sid[u5])
      if refill_g:
        u2 = (u + 2) % _IS
        pltpu.make_async_copy(src_sl(i + 2), ivs.at[u2], sis[u2]).wait()
        pltpu.async_copy(y_hbm.at[ivs.at[u2]], rows.at[(u + 2) % _NB],
                         sg[(u + 2) % _NB])

    for u in range(_IS):
      row_slot(u, u, u == 0, True, True, True)

    @pl.loop(1, CPW // _IS - 1)
    def _(k):
      for u in range(_IS):
        row_slot(k * _IS + u, u, False, True, True, True)

    for u in range(_IS):
      i = (CPW - _IS) + u
      row_slot(i, u, False, i + _IS < CPW, i + _IS - 1 < CPW, i + 2 < CPW)
    scat(CPW - 1).wait()

    plsc.subcore_barrier()
    pltpu.sync_copy(acc.at[pl.ds(s * RPT, RPT)],
                    out_hbm.at[c, pl.ds(s * RPT, RPT)])
    if with_counts:
      pltpu.sync_copy(cnt, cnt_hbm.at[wid])

  return seg_sum


_seg_sum_cnt = _make_seg_sum(True, CE1, CPW1)
_seg_sum = _make_seg_sum(False, CE2, CPW2)


_BLK = 2048
_GRID = N_PAD // _BLK


def _matmul_body(x_ref, w_ref, y_ref):
  y_ref[...] = lax.dot_general(x_ref[...], w_ref[...], _DOT,
                               preferred_element_type=jnp.float32)


def _mid_body(p_ref, c_ref, x_ref, wr_ref, b_ref, wl_ref,
              y2_ref, h_ref, inv_ref):
  sacc = p_ref[0] + p_ref[1]
  cnt = lax.dot_general(c_ref[...], jnp.ones((NW, 1), jnp.float32),
                        (((0,), (0,)), ((), ())),
                        preferred_element_type=jnp.float32)
  inv = 1.0 / jnp.maximum(cnt, 1.0)
  z1 = lax.dot_general(x_ref[...], wr_ref[...], _DOT,
                       preferred_element_type=jnp.float32) + b_ref[...]
  h = jnp.maximum(sacc * inv + z1, 0.0)
  y2_ref[...] = lax.dot_general(h, wl_ref[...], _DOT,
                                preferred_element_type=jnp.float32)
  h_ref[...] = h
  inv_ref[...] = jnp.broadcast_to(inv, (_BLK, D))


def _post_body(p_ref, h_ref, wr_ref, b_ref, inv_ref, o_ref):
  sacc = p_ref[0] + p_ref[1]
  z2 = lax.dot_general(h_ref[...], wr_ref[...], _DOT,
                       preferred_element_type=jnp.float32) + b_ref[...]
  o_ref[...] = sacc * inv_ref[...] + z2


def _row_spec():
  return pl.BlockSpec((_BLK, D), lambda i: (i, 0))


def _full_spec(shape):
  nd = len(shape)
  return pl.BlockSpec(shape, lambda i, _nd=nd: (0,) * _nd)


def _part_spec():
  return pl.BlockSpec((NC, _BLK, D), lambda i: (0, i, 0))


_row_f32 = jax.ShapeDtypeStruct((N_PAD, D), jnp.float32)

_matmul = pl.pallas_call(
    _matmul_body,
    grid=(_GRID,),
    in_specs=[_row_spec(), _full_spec((D, D))],
    out_specs=_row_spec(),
    out_shape=_row_f32,
)

_mid = pl.pallas_call(
    _mid_body,
    grid=(_GRID,),
    in_specs=[_part_spec(), pl.BlockSpec((NW, _BLK), lambda i: (0, i)),
              _row_spec(), _full_spec((D, D)), _full_spec((1, D)),
              _full_spec((D, D))],
    out_specs=[_row_spec(), _row_spec(), _row_spec()],
    out_shape=[_row_f32, _row_f32, _row_f32],
)

_post = pl.pallas_call(
    _post_body,
    grid=(_GRID,),
    in_specs=[_part_spec(), _row_spec(), _full_spec((D, D)),
              _full_spec((1, D)), _row_spec()],
    out_specs=_row_spec(),
    out_shape=jax.ShapeDtypeStruct((N_NODES, D), jnp.float32),
)


def _pad_edges(edge_index, e_pad):
  pad_ar = jnp.arange(e_pad - E_EDGES, dtype=jnp.int32)
  src1 = jnp.concatenate([edge_index[0], pad_ar % N_NODES])
  dst1 = jnp.concatenate([edge_index[1],
                          N_NODES + pad_ar % (N_PAD - N_NODES)])
  return src1, dst1


def kernel(x, edge_index, W1l, b1l, W1r, W2l, b2l, W2r):
  src1a, dst1a = _pad_edges(edge_index, NW * CPW1 * CE1)
  src1b, dst1b = _pad_edges(edge_index, NW * CPW2 * CE2)
  x_pad = jnp.concatenate([x, jnp.zeros((N_PAD - N_NODES, D), x.dtype)])

  y1 = _matmul(x_pad, W1l)
  p1, pc = _seg_sum_cnt(y1, src1a, dst1a)
  y2, h, inv = _mid(p1, pc.reshape(NW, N_PAD), x_pad, W1r,
                    b1l.reshape(1, D), W2l)
  p2, = _seg_sum(y2, src1b, dst1b)
  return _post(p2, h, W2r, b2l.reshape(1, D), inv)
